# baseline (speedup 1.0000x reference)
import functools

import jax
import jax.numpy as jnp
from jax import lax
from jax.experimental import pallas as pl
from jax.experimental.pallas import tpu as pltpu
from jax.experimental.pallas import tpu_sc as plsc

N = 10000
E = 320000
HID = 128
WCOL = 80
NW = 32
CHUNK = 128
CPT = 80
EP = NW * CPT * CHUNK
NPAD = 10016

_f32 = jnp.float32


def _k1_body(x_ref, b_ref, wl_ref, attl_ref, attr_ref, w1a_ref, w1b_ref,
             ub_ref, tin_ref, tout_ref):
    xb = x_ref[...]
    bb = b_ref[...]
    wl = wl_ref[...]
    vl = jnp.dot(attl_ref[...], wl, preferred_element_type=_f32)
    vr = jnp.dot(attr_ref[...], wl, preferred_element_type=_f32)
    dn = (((1,), (1,)), ((), ()))
    al = lax.dot_general(xb, vl, dn, preferred_element_type=_f32)
    ar = lax.dot_general(xb, vr, dn, preferred_element_type=_f32)
    qin = jnp.dot(xb, w1a_ref[...], preferred_element_type=_f32) \
        + bb * ub_ref[0:1, :]
    qout = jnp.dot(xb, w1b_ref[...], preferred_element_type=_f32) \
        + bb * ub_ref[1:2, :]
    pad = jnp.zeros((xb.shape[0], WCOL - 65), dtype=_f32)
    tin_ref[...] = jnp.concatenate([qin, al, pad], axis=1)
    tout_ref[...] = jnp.concatenate([qout, ar, pad], axis=1)


def _k1(x, b, W_l, attl, attr, W1a, W1b, ub):
    BN = 1000
    grid = (N // BN,)
    full = lambda s: pl.BlockSpec(s, lambda i: (0, 0))
    return pl.pallas_call(
        _k1_body,
        grid=grid,
        in_specs=[
            pl.BlockSpec((BN, HID), lambda i: (i, 0)),
            pl.BlockSpec((BN, 1), lambda i: (i, 0)),
            full((HID, HID)),
            full((1, HID)),
            full((1, HID)),
            full((HID, 64)),
            full((HID, 64)),
            full((2, 64)),
        ],
        out_specs=[
            pl.BlockSpec((BN, WCOL), lambda i: (i, 0)),
            pl.BlockSpec((BN, WCOL), lambda i: (i, 0)),
        ],
        out_shape=[
            jax.ShapeDtypeStruct((N, WCOL), _f32),
            jax.ShapeDtypeStruct((N, WCOL), _f32),
        ],
    )(x, b, W_l, attl, attr, W1a, W1b, ub)


def _k2_body(tin_hbm, tout_hbm, src_hbm, dst_hbm, g_hbm,
             idx_a, idx_b,
             buf_a0, buf_a1, buf_b0, buf_b1, buf_o0, buf_o1,
             sem_a0, sem_a1, sem_b0, sem_b1, sem_w0, sem_w1):
    wid = lax.axis_index("c") * 16 + lax.axis_index("s")
    tile_base = wid * (CPT * CHUNK)
    bufs_a = [buf_a0, buf_a1]
    bufs_b = [buf_b0, buf_b1]
    bufs_o = [buf_o0, buf_o1]
    sems_a = [sem_a0, sem_a1]
    sems_b = [sem_b0, sem_b1]
    sems_w = [sem_w0, sem_w1]

    pltpu.sync_copy(src_hbm.at[pl.ds(tile_base, CPT * CHUNK)], idx_a)
    pltpu.sync_copy(dst_hbm.at[pl.ds(tile_base, CPT * CHUNK)], idx_b)

    def gathers(i, s):
        sl = pl.ds(i * CHUNK, CHUNK)
        pltpu.async_copy(tin_hbm.at[idx_a.at[sl]], bufs_a[s], sems_a[s])
        pltpu.async_copy(tout_hbm.at[idx_b.at[sl]], bufs_b[s], sems_b[s])

    for s in range(2):
        gathers(s, s)

    def step(j, carry):
        for s in range(2):
            i = 2 * j + s
            pltpu.make_async_copy(tin_hbm.at[pl.ds(0, CHUNK)],
                                  bufs_a[s], sems_a[s]).wait()
            pltpu.make_async_copy(tout_hbm.at[pl.ds(0, CHUNK)],
                                  bufs_b[s], sems_b[s]).wait()

            @pl.when(j > 0)
            def _():
                pltpu.make_async_copy(bufs_o[s], g_hbm.at[pl.ds(0, CHUNK)],
                                      sems_w[s]).wait()

            ba, bb, bo = bufs_a[s], bufs_b[s], bufs_o[s]

            @plsc.parallel_loop(0, CHUNK, unroll=8)
            def row(r):
                for k in range(WCOL // 16):
                    c = pl.ds(k * 16, 16)
                    bo[r, c] = ba[r, c] + bb[r, c]

            @pl.when(i + 2 < CPT)
            def _():
                gathers(i + 2, s)

            pltpu.async_copy(bufs_o[s],
                             g_hbm.at[pl.ds(tile_base + i * CHUNK, CHUNK)],
                             sems_w[s])
        return carry
    lax.fori_loop(0, CPT // 2, step, 0)
    for s in range(2):
        pltpu.make_async_copy(bufs_o[s], g_hbm.at[pl.ds(0, CHUNK)],
                              sems_w[s]).wait()


def _k2(Tin, Tout, srcp, dstp):
    mesh = plsc.VectorSubcoreMesh(core_axis_name="c", subcore_axis_name="s")
    return pl.kernel(
        _k2_body,
        out_type=jax.ShapeDtypeStruct((EP, WCOL), _f32),
        mesh=mesh,
        scratch_types=[
            pltpu.VMEM((CPT * CHUNK,), jnp.int32),
            pltpu.VMEM((CPT * CHUNK,), jnp.int32),
        ] + [pltpu.VMEM((CHUNK, WCOL), _f32)] * 6 + [
            pltpu.SemaphoreType.DMA,
            pltpu.SemaphoreType.DMA,
            pltpu.SemaphoreType.DMA,
            pltpu.SemaphoreType.DMA,
            pltpu.SemaphoreType.DMA,
            pltpu.SemaphoreType.DMA,
        ],
        compiler_params=pltpu.CompilerParams(use_tc_tiling_on_sc=False),
    )(Tin, Tout, srcp, dstp)


def _k3_body(g_ref, j_ref, uj_ref, b1_ref, w2t_ref, b2_ref, h_ref):
    g = g_ref[...]
    h1 = jnp.maximum(g[:, :64] + j_ref[...] * uj_ref[...] + b1_ref[...], 0.0)
    a = g[:, 64:65]
    a = jnp.where(a > 0, a, 0.2 * a)
    w = jnp.exp(a)
    h2 = jnp.dot(w * h1, w2t_ref[...], preferred_element_type=_f32) \
        + w * b2_ref[...]
    h2 = jnp.maximum(h2, 0.0)
    pad = jnp.zeros((g.shape[0], WCOL - 65), dtype=_f32)
    h_ref[...] = jnp.concatenate([h2, w, pad], axis=1)


def _k3(G, Jp, uJ, b1, W2t, b2):
    BE = 1024
    grid = (EP // BE + (1 if EP % BE else 0),)
    full = lambda s: pl.BlockSpec(s, lambda i: (0, 0))
    return pl.pallas_call(
        _k3_body,
        grid=grid,
        in_specs=[
            pl.BlockSpec((BE, WCOL), lambda i: (i, 0)),
            pl.BlockSpec((BE, 1), lambda i: (i, 0)),
            full((1, 64)),
            full((1, 64)),
            full((64, 64)),
            full((1, 64)),
        ],
        out_specs=pl.BlockSpec((BE, WCOL), lambda i: (i, 0)),
        out_shape=jax.ShapeDtypeStruct((EP, WCOL), _f32),
    )(G, Jp, uJ, b1, W2t, b2)


def _k4_body(h_hbm, dst_hbm, acc_hbm, idx_v, rows_v, rows_v2, zbuf, acc,
             sem, sem2):
    cid = lax.axis_index("c")
    sid = lax.axis_index("s")
    wid = cid * 16 + sid
    rpt = NPAD // 16

    def zrow(r, c2):
        for k in range(WCOL // 16):
            zbuf[r, pl.ds(k * 16, 16)] = jnp.zeros((16,), _f32)
        return c2
    lax.fori_loop(0, rpt, zrow, 0)
    pltpu.sync_copy(zbuf, acc.at[pl.ds(sid * rpt, rpt)])
    plsc.subcore_barrier()

    tile_base = wid * (CPT * CHUNK)
    bufs = [rows_v, rows_v2]
    sems = [sem, sem2]
    pltpu.sync_copy(dst_hbm.at[pl.ds(wid * CPT, CPT)], idx_v)

    def load(i, s):
        pltpu.async_copy(h_hbm.at[pl.ds(tile_base + i * CHUNK, CHUNK)],
                         bufs[s], sems[s])

    for s in range(2):
        load(s, s)

    def step(j, carry):
        for s in range(2):
            i = 2 * j + s
            pltpu.make_async_copy(h_hbm.at[pl.ds(0, CHUNK)],
                                  bufs[s], sems[s]).wait()
            pltpu.sync_copy(bufs[s], acc.at[idx_v.at[i]], add=True)

            @pl.when(i + 2 < CPT)
            def _():
                load(i + 2, s)
        return carry
    lax.fori_loop(0, CPT // 2, step, 0)

    plsc.subcore_barrier()
    pltpu.sync_copy(acc.at[pl.ds(sid * rpt, rpt)], zbuf)
    pltpu.sync_copy(zbuf, acc_hbm.at[cid, pl.ds(sid * rpt, rpt)])


def _k4(H, dst_scat):
    mesh = plsc.VectorSubcoreMesh(core_axis_name="c", subcore_axis_name="s")
    return pl.kernel(
        _k4_body,
        out_type=jax.ShapeDtypeStruct((2, NPAD, WCOL), _f32),
        mesh=mesh,
        scratch_types=[
            pltpu.VMEM((CPT, CHUNK), jnp.int32),
            pltpu.VMEM((CHUNK, WCOL), _f32),
            pltpu.VMEM((CHUNK, WCOL), _f32),
            pltpu.VMEM((NPAD // 16, WCOL), _f32),
            pltpu.VMEM_SHARED((NPAD, WCOL), _f32),
            pltpu.SemaphoreType.DMA,
            pltpu.SemaphoreType.DMA,
        ],
        compiler_params=pltpu.CompilerParams(use_tc_tiling_on_sc=False),
    )(H, dst_scat)


def _k5_body(acc_ref, x_ref, w3t_ref, b3_ref, wiht_ref, bih_ref,
             whht_ref, bhh_ref, gb_ref, out_ref):
    a = acc_ref[0] + acc_ref[1]
    xb = x_ref[...]
    ws = a[:, 64:65]
    inv = 1.0 / (ws + 1e-16)
    salpha = ws * inv
    agg = jnp.dot(a[:, :64] * inv, w3t_ref[...], preferred_element_type=_f32) \
        + b3_ref[...] * salpha
    gi = jnp.dot(agg, wiht_ref[...], preferred_element_type=_f32) + bih_ref[...]
    gh = jnp.dot(xb, whht_ref[...], preferred_element_type=_f32) + bhh_ref[...]
    r = jax.nn.sigmoid(gi[:, :HID] + gh[:, :HID])
    z = jax.nn.sigmoid(gi[:, HID:2 * HID] + gh[:, HID:2 * HID])
    n = jnp.tanh(gi[:, 2 * HID:] + r * gh[:, 2 * HID:])
    out_ref[...] = (1.0 - z) * n + z * xb + gb_ref[...]


def _k5(acc, x, W3t, b3, Wiht, bih, Whht, bhh, gbias):
    BN = 1000
    grid = (N // BN,)
    full = lambda s: pl.BlockSpec(s, lambda i: tuple(0 for _ in s))
    return pl.pallas_call(
        _k5_body,
        grid=grid,
        in_specs=[
            pl.BlockSpec((2, BN, WCOL), lambda i: (0, i, 0)),
            pl.BlockSpec((BN, HID), lambda i: (i, 0)),
            full((64, HID)),
            full((1, HID)),
            full((HID, 3 * HID)),
            full((1, 3 * HID)),
            full((HID, 3 * HID)),
            full((1, 3 * HID)),
            full((1, HID)),
        ],
        out_specs=pl.BlockSpec((BN, HID), lambda i: (i, 0)),
        out_shape=jax.ShapeDtypeStruct((N, HID), _f32),
    )(acc, x, W3t, b3, Wiht, bih, Whht, bhh, gbias)


def kernel(x, edge_index, J_msg, b, idx_msg_edge, W_l, att_l, att_r, gat_bias,
           msg_W1, msg_b1, msg_W2, msg_b2, msg_W3, msg_b3,
           gru_Wih, gru_bih, gru_Whh, gru_bhh):
    del idx_msg_edge

    attl = att_l.reshape(1, HID)
    attr = att_r.reshape(1, HID)
    W1a = msg_W1[:, :HID].T
    W1b = msg_W1[:, HID + 4:2 * HID + 4].T
    ub = jnp.stack([msg_W1[:, 128] - msg_W1[:, 129],
                    msg_W1[:, 261] - msg_W1[:, 260]], axis=0)
    uJ = (msg_W1[:, 130] - msg_W1[:, 131]
          + msg_W1[:, 263] - msg_W1[:, 262]).reshape(1, 64)
    b1 = msg_b1.reshape(1, 64)
    b2 = msg_b2.reshape(1, 64)
    b3 = msg_b3.reshape(1, HID)
    W2t = msg_W2.T
    W3t = msg_W3.T
    Wiht = gru_Wih.T
    Whht = gru_Whh.T
    bih = gru_bih.reshape(1, 3 * HID)
    bhh = gru_bhh.reshape(1, 3 * HID)
    gbias = gat_bias.reshape(1, HID)

    src = edge_index[:, 0].astype(jnp.int32)
    dst = edge_index[:, 1].astype(jnp.int32)
    npad = EP - E
    srcp = jnp.concatenate([src, jnp.zeros((npad,), jnp.int32)])
    dstp = jnp.concatenate([dst, jnp.zeros((npad,), jnp.int32)])
    dst_scat = jnp.concatenate(
        [dst, jnp.full((npad,), N, jnp.int32)]).reshape(NW * CPT, CHUNK)
    Jp = jnp.concatenate([J_msg, jnp.zeros((npad, 1), _f32)], axis=0)

    Tin, Tout = _k1(x, b, W_l, attl, attr, W1a, W1b, ub)
    G = _k2(Tin, Tout, srcp, dstp)
    H = _k3(G, Jp, uJ, b1, W2t, b2)
    acc = _k4(H, dst_scat)
    out = _k5(acc[:, :N, :], x, W3t, b3, Wiht, bih, Whht, bhh, gbias)
    return out


# speedup vs baseline: 1.1983x; 1.1983x over previous
import functools

import jax
import jax.numpy as jnp
from jax import lax
from jax.experimental import pallas as pl
from jax.experimental.pallas import tpu as pltpu
from jax.experimental.pallas import tpu_sc as plsc

N = 10000
E = 320000
HID = 128
WCOL = 80
NW = 32
CHUNK = 112
CPT = 90
EP = NW * CPT * CHUNK
NPAD = 10016

_f32 = jnp.float32


def _k1_body(x_ref, b_ref, wl_ref, attl_ref, attr_ref, w1a_ref, w1b_ref,
             ub_ref, tin_ref, tout_ref):
    xb = x_ref[...]
    bb = b_ref[...]
    wl = wl_ref[...]
    vl = jnp.dot(attl_ref[...], wl, preferred_element_type=_f32)
    vr = jnp.dot(attr_ref[...], wl, preferred_element_type=_f32)
    dn = (((1,), (1,)), ((), ()))
    al = lax.dot_general(xb, vl, dn, preferred_element_type=_f32)
    ar = lax.dot_general(xb, vr, dn, preferred_element_type=_f32)
    qin = jnp.dot(xb, w1a_ref[...], preferred_element_type=_f32) \
        + bb * ub_ref[0:1, :]
    qout = jnp.dot(xb, w1b_ref[...], preferred_element_type=_f32) \
        + bb * ub_ref[1:2, :]
    pad = jnp.zeros((xb.shape[0], WCOL - 65), dtype=_f32)
    tin_ref[...] = jnp.concatenate([qin, al, pad], axis=1)
    tout_ref[...] = jnp.concatenate([qout, ar, pad], axis=1)


def _k1(x, b, W_l, attl, attr, W1a, W1b, ub):
    BN = 1000
    grid = (N // BN,)
    full = lambda s: pl.BlockSpec(s, lambda i: (0, 0))
    return pl.pallas_call(
        _k1_body,
        grid=grid,
        in_specs=[
            pl.BlockSpec((BN, HID), lambda i: (i, 0)),
            pl.BlockSpec((BN, 1), lambda i: (i, 0)),
            full((HID, HID)),
            full((1, HID)),
            full((1, HID)),
            full((HID, 64)),
            full((HID, 64)),
            full((2, 64)),
        ],
        out_specs=[
            pl.BlockSpec((BN, WCOL), lambda i: (i, 0)),
            pl.BlockSpec((BN, WCOL), lambda i: (i, 0)),
        ],
        out_shape=[
            jax.ShapeDtypeStruct((N, WCOL), _f32),
            jax.ShapeDtypeStruct((N, WCOL), _f32),
        ],
    )(x, b, W_l, attl, attr, W1a, W1b, ub)


def _k2_body(tin_hbm, tout_hbm, src_hbm, dst_hbm, g_hbm,
             idx_a, idx_b,
             buf_a0, buf_a1, buf_b0, buf_b1, buf_o0, buf_o1,
             tout_s,
             sem_a0, sem_a1, sem_b0, sem_b1, sem_w0, sem_w1):
    sid = lax.axis_index("s")
    wid = lax.axis_index("c") * 16 + sid
    tile_base = wid * (CPT * CHUNK)

    rps = N // 16
    st = pl.ds(sid * rps, rps)
    pltpu.sync_copy(tout_hbm.at[st], tout_s.at[st])
    plsc.subcore_barrier()
    bufs_a = [buf_a0, buf_a1]
    bufs_b = [buf_b0, buf_b1]
    bufs_o = [buf_o0, buf_o1]
    sems_a = [sem_a0, sem_a1]
    sems_b = [sem_b0, sem_b1]
    sems_w = [sem_w0, sem_w1]

    pltpu.sync_copy(src_hbm.at[pl.ds(tile_base, CPT * CHUNK)], idx_a)
    pltpu.sync_copy(dst_hbm.at[pl.ds(tile_base, CPT * CHUNK)], idx_b)

    def gathers(i, s):
        sl = pl.ds(i * CHUNK, CHUNK)
        pltpu.async_copy(tin_hbm.at[idx_a.at[sl]], bufs_a[s], sems_a[s])
        pltpu.async_copy(tout_s.at[idx_b.at[sl]], bufs_b[s], sems_b[s])

    for s in range(2):
        gathers(s, s)

    def step(j, carry):
        for s in range(2):
            i = 2 * j + s
            pltpu.make_async_copy(tin_hbm.at[pl.ds(0, CHUNK)],
                                  bufs_a[s], sems_a[s]).wait()
            pltpu.make_async_copy(tin_hbm.at[pl.ds(0, CHUNK)],
                                  bufs_b[s], sems_b[s]).wait()

            @pl.when(j > 0)
            def _():
                pltpu.make_async_copy(bufs_o[s], g_hbm.at[pl.ds(0, CHUNK)],
                                      sems_w[s]).wait()

            ba, bb, bo = bufs_a[s], bufs_b[s], bufs_o[s]

            @plsc.parallel_loop(0, CHUNK, unroll=8)
            def row(r):
                for k in range(WCOL // 16):
                    c = pl.ds(k * 16, 16)
                    bo[r, c] = ba[r, c] + bb[r, c]

            @pl.when(i + 2 < CPT)
            def _():
                gathers(i + 2, s)

            pltpu.async_copy(bufs_o[s],
                             g_hbm.at[pl.ds(tile_base + i * CHUNK, CHUNK)],
                             sems_w[s])
        return carry
    lax.fori_loop(0, CPT // 2, step, 0)
    for s in range(2):
        pltpu.make_async_copy(bufs_o[s], g_hbm.at[pl.ds(0, CHUNK)],
                              sems_w[s]).wait()


def _k2(Tin, Tout, srcp, dstp):
    mesh = plsc.VectorSubcoreMesh(core_axis_name="c", subcore_axis_name="s")
    return pl.kernel(
        _k2_body,
        out_type=jax.ShapeDtypeStruct((EP, WCOL), _f32),
        mesh=mesh,
        scratch_types=[
            pltpu.VMEM((CPT * CHUNK,), jnp.int32),
            pltpu.VMEM((CPT * CHUNK,), jnp.int32),
        ] + [pltpu.VMEM((CHUNK, WCOL), _f32)] * 6 + [
            pltpu.VMEM_SHARED((NPAD, WCOL), _f32),
            pltpu.SemaphoreType.DMA,
            pltpu.SemaphoreType.DMA,
            pltpu.SemaphoreType.DMA,
            pltpu.SemaphoreType.DMA,
            pltpu.SemaphoreType.DMA,
            pltpu.SemaphoreType.DMA,
        ],
        compiler_params=pltpu.CompilerParams(use_tc_tiling_on_sc=False),
    )(Tin, Tout, srcp, dstp)


def _k3_body(g_ref, j_ref, uj_ref, b1_ref, w2t_ref, b2_ref, h_ref):
    g = g_ref[...]
    h1 = jnp.maximum(g[:, :64] + j_ref[...] * uj_ref[...] + b1_ref[...], 0.0)
    a = g[:, 64:65]
    a = jnp.where(a > 0, a, 0.2 * a)
    w = jnp.exp(a)
    h2 = jnp.dot(w * h1, w2t_ref[...], preferred_element_type=_f32) \
        + w * b2_ref[...]
    h2 = jnp.maximum(h2, 0.0)
    pad = jnp.zeros((g.shape[0], WCOL - 65), dtype=_f32)
    h_ref[...] = jnp.concatenate([h2, w, pad], axis=1)


def _k3(G, Jp, uJ, b1, W2t, b2):
    BE = 1024
    grid = (EP // BE + (1 if EP % BE else 0),)
    full = lambda s: pl.BlockSpec(s, lambda i: (0, 0))
    return pl.pallas_call(
        _k3_body,
        grid=grid,
        in_specs=[
            pl.BlockSpec((BE, WCOL), lambda i: (i, 0)),
            pl.BlockSpec((BE, 1), lambda i: (i, 0)),
            full((1, 64)),
            full((1, 64)),
            full((64, 64)),
            full((1, 64)),
        ],
        out_specs=pl.BlockSpec((BE, WCOL), lambda i: (i, 0)),
        out_shape=jax.ShapeDtypeStruct((EP, WCOL), _f32),
    )(G, Jp, uJ, b1, W2t, b2)


def _k4_body(h_hbm, dst_hbm, acc_hbm, idx_v, rows_v, rows_v2, zbuf, acc,
             sem, sem2):
    cid = lax.axis_index("c")
    sid = lax.axis_index("s")
    wid = cid * 16 + sid
    rpt = NPAD // 16

    def zrow(r, c2):
        for k in range(WCOL // 16):
            zbuf[r, pl.ds(k * 16, 16)] = jnp.zeros((16,), _f32)
        return c2
    lax.fori_loop(0, rpt, zrow, 0)
    pltpu.sync_copy(zbuf, acc.at[pl.ds(sid * rpt, rpt)])
    plsc.subcore_barrier()

    tile_base = wid * (CPT * CHUNK)
    bufs = [rows_v, rows_v2]
    sems = [sem, sem2]
    pltpu.sync_copy(dst_hbm.at[pl.ds(wid * CPT, CPT)], idx_v)

    def load(i, s):
        pltpu.async_copy(h_hbm.at[pl.ds(tile_base + i * CHUNK, CHUNK)],
                         bufs[s], sems[s])

    for s in range(2):
        load(s, s)

    def step(j, carry):
        for s in range(2):
            i = 2 * j + s
            pltpu.make_async_copy(h_hbm.at[pl.ds(0, CHUNK)],
                                  bufs[s], sems[s]).wait()
            pltpu.sync_copy(bufs[s], acc.at[idx_v.at[i]], add=True)

            @pl.when(i + 2 < CPT)
            def _():
                load(i + 2, s)
        return carry
    lax.fori_loop(0, CPT // 2, step, 0)

    plsc.subcore_barrier()
    pltpu.sync_copy(acc.at[pl.ds(sid * rpt, rpt)], zbuf)
    pltpu.sync_copy(zbuf, acc_hbm.at[cid, pl.ds(sid * rpt, rpt)])


def _k4(H, dst_scat):
    mesh = plsc.VectorSubcoreMesh(core_axis_name="c", subcore_axis_name="s")
    return pl.kernel(
        _k4_body,
        out_type=jax.ShapeDtypeStruct((2, NPAD, WCOL), _f32),
        mesh=mesh,
        scratch_types=[
            pltpu.VMEM((CPT, CHUNK), jnp.int32),
            pltpu.VMEM((CHUNK, WCOL), _f32),
            pltpu.VMEM((CHUNK, WCOL), _f32),
            pltpu.VMEM((NPAD // 16, WCOL), _f32),
            pltpu.VMEM_SHARED((NPAD, WCOL), _f32),
            pltpu.SemaphoreType.DMA,
            pltpu.SemaphoreType.DMA,
        ],
        compiler_params=pltpu.CompilerParams(use_tc_tiling_on_sc=False),
    )(H, dst_scat)


def _k5_body(acc_ref, x_ref, w3t_ref, b3_ref, wiht_ref, bih_ref,
             whht_ref, bhh_ref, gb_ref, out_ref):
    a = acc_ref[0] + acc_ref[1]
    xb = x_ref[...]
    ws = a[:, 64:65]
    inv = 1.0 / (ws + 1e-16)
    salpha = ws * inv
    agg = jnp.dot(a[:, :64] * inv, w3t_ref[...], preferred_element_type=_f32) \
        + b3_ref[...] * salpha
    gi = jnp.dot(agg, wiht_ref[...], preferred_element_type=_f32) + bih_ref[...]
    gh = jnp.dot(xb, whht_ref[...], preferred_element_type=_f32) + bhh_ref[...]
    r = jax.nn.sigmoid(gi[:, :HID] + gh[:, :HID])
    z = jax.nn.sigmoid(gi[:, HID:2 * HID] + gh[:, HID:2 * HID])
    n = jnp.tanh(gi[:, 2 * HID:] + r * gh[:, 2 * HID:])
    out_ref[...] = (1.0 - z) * n + z * xb + gb_ref[...]


def _k5(acc, x, W3t, b3, Wiht, bih, Whht, bhh, gbias):
    BN = 1000
    grid = (N // BN,)
    full = lambda s: pl.BlockSpec(s, lambda i: tuple(0 for _ in s))
    return pl.pallas_call(
        _k5_body,
        grid=grid,
        in_specs=[
            pl.BlockSpec((2, BN, WCOL), lambda i: (0, i, 0)),
            pl.BlockSpec((BN, HID), lambda i: (i, 0)),
            full((64, HID)),
            full((1, HID)),
            full((HID, 3 * HID)),
            full((1, 3 * HID)),
            full((HID, 3 * HID)),
            full((1, 3 * HID)),
            full((1, HID)),
        ],
        out_specs=pl.BlockSpec((BN, HID), lambda i: (i, 0)),
        out_shape=jax.ShapeDtypeStruct((N, HID), _f32),
    )(acc, x, W3t, b3, Wiht, bih, Whht, bhh, gbias)


def kernel(x, edge_index, J_msg, b, idx_msg_edge, W_l, att_l, att_r, gat_bias,
           msg_W1, msg_b1, msg_W2, msg_b2, msg_W3, msg_b3,
           gru_Wih, gru_bih, gru_Whh, gru_bhh):
    del idx_msg_edge

    attl = att_l.reshape(1, HID)
    attr = att_r.reshape(1, HID)
    W1a = msg_W1[:, :HID].T
    W1b = msg_W1[:, HID + 4:2 * HID + 4].T
    ub = jnp.stack([msg_W1[:, 128] - msg_W1[:, 129],
                    msg_W1[:, 261] - msg_W1[:, 260]], axis=0)
    uJ = (msg_W1[:, 130] - msg_W1[:, 131]
          + msg_W1[:, 263] - msg_W1[:, 262]).reshape(1, 64)
    b1 = msg_b1.reshape(1, 64)
    b2 = msg_b2.reshape(1, 64)
    b3 = msg_b3.reshape(1, HID)
    W2t = msg_W2.T
    W3t = msg_W3.T
    Wiht = gru_Wih.T
    Whht = gru_Whh.T
    bih = gru_bih.reshape(1, 3 * HID)
    bhh = gru_bhh.reshape(1, 3 * HID)
    gbias = gat_bias.reshape(1, HID)

    src = edge_index[:, 0].astype(jnp.int32)
    dst = edge_index[:, 1].astype(jnp.int32)
    npad = EP - E
    srcp = jnp.concatenate([src, jnp.zeros((npad,), jnp.int32)])
    dstp = jnp.concatenate([dst, jnp.zeros((npad,), jnp.int32)])
    dst_scat = jnp.concatenate(
        [dst, jnp.full((npad,), N, jnp.int32)]).reshape(NW * CPT, CHUNK)
    Jp = jnp.concatenate([J_msg, jnp.zeros((npad, 1), _f32)], axis=0)

    Tin, Tout = _k1(x, b, W_l, attl, attr, W1a, W1b, ub)
    G = _k2(Tin, Tout, srcp, dstp)
    H = _k3(G, Jp, uJ, b1, W2t, b2)
    acc = _k4(H, dst_scat)
    out = _k5(acc[:, :N, :], x, W3t, b3, Wiht, bih, Whht, bhh, gbias)
    return out


# speedup vs baseline: 1.2092x; 1.0091x over previous
import functools

import jax
import jax.numpy as jnp
from jax import lax
from jax.experimental import pallas as pl
from jax.experimental.pallas import tpu as pltpu
from jax.experimental.pallas import tpu_sc as plsc

N = 10000
E = 320000
HID = 128
WCOL = 80
NW = 32
CHUNK = 112
CPT = 90
EP = NW * CPT * CHUNK
NPAD = 10016

_f32 = jnp.float32


def _k1_body(x_ref, b_ref, wl_ref, attl_ref, attr_ref, w1t_ref,
             tin_ref, tout_ref):
    xb = x_ref[...]
    bb = b_ref[...]
    wl = wl_ref[...]
    w1t = w1t_ref[...]
    vl = jnp.dot(attl_ref[...], wl, preferred_element_type=_f32)
    vr = jnp.dot(attr_ref[...], wl, preferred_element_type=_f32)
    dn = (((1,), (1,)), ((), ()))
    al = lax.dot_general(xb, vl, dn, preferred_element_type=_f32)
    ar = lax.dot_general(xb, vr, dn, preferred_element_type=_f32)
    qin = jnp.dot(xb, w1t[:HID, :], preferred_element_type=_f32) \
        + bb * (w1t[128:129, :] - w1t[129:130, :])
    qout = jnp.dot(xb, w1t[HID + 4:2 * HID + 4, :], preferred_element_type=_f32) \
        + bb * (w1t[261:262, :] - w1t[260:261, :])
    pad = jnp.zeros((xb.shape[0], WCOL - 65), dtype=_f32)
    tin_ref[...] = jnp.concatenate([qin, al, pad], axis=1)
    tout_ref[...] = jnp.concatenate([qout, ar, pad], axis=1)


def _k1(x, b, W_l, attl, attr, W1T):
    BN = 1000
    grid = (N // BN,)
    full = lambda s: pl.BlockSpec(s, lambda i: (0, 0))
    return pl.pallas_call(
        _k1_body,
        grid=grid,
        in_specs=[
            pl.BlockSpec((BN, HID), lambda i: (i, 0)),
            pl.BlockSpec((BN, 1), lambda i: (i, 0)),
            full((HID, HID)),
            full((1, HID)),
            full((1, HID)),
            full((2 * HID + 8, 64)),
        ],
        out_specs=[
            pl.BlockSpec((BN, WCOL), lambda i: (i, 0)),
            pl.BlockSpec((BN, WCOL), lambda i: (i, 0)),
        ],
        out_shape=[
            jax.ShapeDtypeStruct((N, WCOL), _f32),
            jax.ShapeDtypeStruct((N, WCOL), _f32),
        ],
    )(x, b, W_l, attl, attr, W1T)


def _k2_body(tin_hbm, tout_hbm, src_hbm, dst_hbm, g_hbm,
             idx_a, idx_b,
             buf_a0, buf_a1, buf_b0, buf_b1, buf_o0, buf_o1,
             tout_s,
             sem_a0, sem_a1, sem_b0, sem_b1, sem_w0, sem_w1):
    sid = lax.axis_index("s")
    wid = lax.axis_index("c") * 16 + sid
    tile_base = wid * (CPT * CHUNK)

    rps = N // 16
    st = pl.ds(sid * rps, rps)
    pltpu.sync_copy(tout_hbm.at[st], tout_s.at[st])
    plsc.subcore_barrier()
    bufs_a = [buf_a0, buf_a1]
    bufs_b = [buf_b0, buf_b1]
    bufs_o = [buf_o0, buf_o1]
    sems_a = [sem_a0, sem_a1]
    sems_b = [sem_b0, sem_b1]
    sems_w = [sem_w0, sem_w1]

    pltpu.sync_copy(src_hbm.at[pl.ds(tile_base, CPT * CHUNK)], idx_a)
    pltpu.sync_copy(dst_hbm.at[pl.ds(tile_base, CPT * CHUNK)], idx_b)

    def gathers(i, s):
        sl = pl.ds(i * CHUNK, CHUNK)
        pltpu.async_copy(tin_hbm.at[idx_a.at[sl]], bufs_a[s], sems_a[s])
        pltpu.async_copy(tout_s.at[idx_b.at[sl]], bufs_b[s], sems_b[s])

    for s in range(2):
        gathers(s, s)

    def step(j, carry):
        for s in range(2):
            i = 2 * j + s
            pltpu.make_async_copy(tin_hbm.at[pl.ds(0, CHUNK)],
                                  bufs_a[s], sems_a[s]).wait()
            pltpu.make_async_copy(tin_hbm.at[pl.ds(0, CHUNK)],
                                  bufs_b[s], sems_b[s]).wait()

            @pl.when(j > 0)
            def _():
                pltpu.make_async_copy(bufs_o[s], g_hbm.at[pl.ds(0, CHUNK)],
                                      sems_w[s]).wait()

            ba, bb, bo = bufs_a[s], bufs_b[s], bufs_o[s]

            @plsc.parallel_loop(0, CHUNK, unroll=8)
            def row(r):
                for k in range(WCOL // 16):
                    c = pl.ds(k * 16, 16)
                    bo[r, c] = ba[r, c] + bb[r, c]

            @pl.when(i + 2 < CPT)
            def _():
                gathers(i + 2, s)

            pltpu.async_copy(bufs_o[s],
                             g_hbm.at[pl.ds(tile_base + i * CHUNK, CHUNK)],
                             sems_w[s])
        return carry
    lax.fori_loop(0, CPT // 2, step, 0)
    for s in range(2):
        pltpu.make_async_copy(bufs_o[s], g_hbm.at[pl.ds(0, CHUNK)],
                              sems_w[s]).wait()


def _k2(Tin, Tout, srcp, dstp):
    mesh = plsc.VectorSubcoreMesh(core_axis_name="c", subcore_axis_name="s")
    return pl.kernel(
        _k2_body,
        out_type=jax.ShapeDtypeStruct((EP, WCOL), _f32),
        mesh=mesh,
        scratch_types=[
            pltpu.VMEM((CPT * CHUNK,), jnp.int32),
            pltpu.VMEM((CPT * CHUNK,), jnp.int32),
        ] + [pltpu.VMEM((CHUNK, WCOL), _f32)] * 6 + [
            pltpu.VMEM_SHARED((NPAD, WCOL), _f32),
            pltpu.SemaphoreType.DMA,
            pltpu.SemaphoreType.DMA,
            pltpu.SemaphoreType.DMA,
            pltpu.SemaphoreType.DMA,
            pltpu.SemaphoreType.DMA,
            pltpu.SemaphoreType.DMA,
        ],
        compiler_params=pltpu.CompilerParams(use_tc_tiling_on_sc=False),
    )(Tin, Tout, srcp, dstp)


def _k3_body(g_ref, j_ref, w1t_ref, b1_ref, w2_ref, b2_ref, h_ref):
    g = g_ref[...]
    w1t = w1t_ref[...]
    uj = (w1t[130:131, :] - w1t[131:132, :]
          + w1t[263:264, :] - w1t[262:263, :])
    h1 = jnp.maximum(g[:, :64] + j_ref[...] * uj + b1_ref[...], 0.0)
    a = g[:, 64:65]
    a = jnp.where(a > 0, a, 0.2 * a)
    w = jnp.exp(a)
    dn = (((1,), (1,)), ((), ()))
    h2 = lax.dot_general(w * h1, w2_ref[...], dn, preferred_element_type=_f32) \
        + w * b2_ref[...]
    h2 = jnp.maximum(h2, 0.0)
    pad = jnp.zeros((g.shape[0], WCOL - 65), dtype=_f32)
    h_ref[...] = jnp.concatenate([h2, w, pad], axis=1)


def _k3(G, Jp, W1T, b1, W2, b2):
    BE = 1024
    grid = (EP // BE + (1 if EP % BE else 0),)
    full = lambda s: pl.BlockSpec(s, lambda i: (0, 0))
    return pl.pallas_call(
        _k3_body,
        grid=grid,
        in_specs=[
            pl.BlockSpec((BE, WCOL), lambda i: (i, 0)),
            pl.BlockSpec((BE, 1), lambda i: (i, 0)),
            full((2 * HID + 8, 64)),
            full((1, 64)),
            full((64, 64)),
            full((1, 64)),
        ],
        out_specs=pl.BlockSpec((BE, WCOL), lambda i: (i, 0)),
        out_shape=jax.ShapeDtypeStruct((EP, WCOL), _f32),
    )(G, Jp, W1T, b1, W2, b2)


def _k4_body(h_hbm, dst_hbm, acc_hbm, idx_v, rows_v, rows_v2, zbuf, acc,
             sem, sem2):
    cid = lax.axis_index("c")
    sid = lax.axis_index("s")
    wid = cid * 16 + sid
    rpt = NPAD // 16

    def zrow(r, c2):
        for k in range(WCOL // 16):
            zbuf[r, pl.ds(k * 16, 16)] = jnp.zeros((16,), _f32)
        return c2
    lax.fori_loop(0, rpt, zrow, 0)
    pltpu.sync_copy(zbuf, acc.at[pl.ds(sid * rpt, rpt)])
    plsc.subcore_barrier()

    tile_base = wid * (CPT * CHUNK)
    bufs = [rows_v, rows_v2]
    sems = [sem, sem2]
    pltpu.sync_copy(dst_hbm.at[pl.ds(wid * CPT, CPT)], idx_v)

    def load(i, s):
        pltpu.async_copy(h_hbm.at[pl.ds(tile_base + i * CHUNK, CHUNK)],
                         bufs[s], sems[s])

    for s in range(2):
        load(s, s)

    def step(j, carry):
        for s in range(2):
            i = 2 * j + s
            pltpu.make_async_copy(h_hbm.at[pl.ds(0, CHUNK)],
                                  bufs[s], sems[s]).wait()
            pltpu.sync_copy(bufs[s], acc.at[idx_v.at[i]], add=True)

            @pl.when(i + 2 < CPT)
            def _():
                load(i + 2, s)
        return carry
    lax.fori_loop(0, CPT // 2, step, 0)

    plsc.subcore_barrier()
    pltpu.sync_copy(acc.at[pl.ds(sid * rpt, rpt)], zbuf)
    pltpu.sync_copy(zbuf, acc_hbm.at[cid, pl.ds(sid * rpt, rpt)])


def _k4(H, dst_scat):
    mesh = plsc.VectorSubcoreMesh(core_axis_name="c", subcore_axis_name="s")
    return pl.kernel(
        _k4_body,
        out_type=jax.ShapeDtypeStruct((2, NPAD, WCOL), _f32),
        mesh=mesh,
        scratch_types=[
            pltpu.VMEM((CPT, CHUNK), jnp.int32),
            pltpu.VMEM((CHUNK, WCOL), _f32),
            pltpu.VMEM((CHUNK, WCOL), _f32),
            pltpu.VMEM((NPAD // 16, WCOL), _f32),
            pltpu.VMEM_SHARED((NPAD, WCOL), _f32),
            pltpu.SemaphoreType.DMA,
            pltpu.SemaphoreType.DMA,
        ],
        compiler_params=pltpu.CompilerParams(use_tc_tiling_on_sc=False),
    )(H, dst_scat)


def _k5_body(acc_ref, x_ref, w3_ref, b3_ref, wih_ref, bih_ref,
             whh_ref, bhh_ref, gb_ref, out_ref):
    a = acc_ref[0] + acc_ref[1]
    xb = x_ref[...]
    ws = a[:, 64:65]
    inv = 1.0 / (ws + 1e-16)
    salpha = ws * inv
    dn = (((1,), (1,)), ((), ()))
    agg = lax.dot_general(a[:, :64] * inv, w3_ref[...], dn,
                          preferred_element_type=_f32) \
        + b3_ref[...] * salpha
    gi = lax.dot_general(agg, wih_ref[...], dn,
                         preferred_element_type=_f32) + bih_ref[...]
    gh = lax.dot_general(xb, whh_ref[...], dn,
                         preferred_element_type=_f32) + bhh_ref[...]
    r = jax.nn.sigmoid(gi[:, :HID] + gh[:, :HID])
    z = jax.nn.sigmoid(gi[:, HID:2 * HID] + gh[:, HID:2 * HID])
    n = jnp.tanh(gi[:, 2 * HID:] + r * gh[:, 2 * HID:])
    out_ref[...] = (1.0 - z) * n + z * xb + gb_ref[...]


def _k5(acc, x, W3, b3, Wih, bih, Whh, bhh, gbias):
    BN = 1000
    grid = (N // BN,)
    full = lambda s: pl.BlockSpec(s, lambda i: tuple(0 for _ in s))
    return pl.pallas_call(
        _k5_body,
        grid=grid,
        in_specs=[
            pl.BlockSpec((2, BN, WCOL), lambda i: (0, i, 0)),
            pl.BlockSpec((BN, HID), lambda i: (i, 0)),
            full((HID, 64)),
            full((1, HID)),
            full((3 * HID, HID)),
            full((1, 3 * HID)),
            full((3 * HID, HID)),
            full((1, 3 * HID)),
            full((1, HID)),
        ],
        out_specs=pl.BlockSpec((BN, HID), lambda i: (i, 0)),
        out_shape=jax.ShapeDtypeStruct((N, HID), _f32),
    )(acc, x, W3, b3, Wih, bih, Whh, bhh, gbias)


def kernel(x, edge_index, J_msg, b, idx_msg_edge, W_l, att_l, att_r, gat_bias,
           msg_W1, msg_b1, msg_W2, msg_b2, msg_W3, msg_b3,
           gru_Wih, gru_bih, gru_Whh, gru_bhh):
    del idx_msg_edge

    attl = att_l.reshape(1, HID)
    attr = att_r.reshape(1, HID)
    W1T = msg_W1.T
    b1 = msg_b1.reshape(1, 64)
    b2 = msg_b2.reshape(1, 64)
    b3 = msg_b3.reshape(1, HID)
    bih = gru_bih.reshape(1, 3 * HID)
    bhh = gru_bhh.reshape(1, 3 * HID)
    gbias = gat_bias.reshape(1, HID)

    src = edge_index[:, 0].astype(jnp.int32)
    dst = edge_index[:, 1].astype(jnp.int32)
    npad = EP - E
    srcp = jnp.concatenate([src, jnp.zeros((npad,), jnp.int32)])
    dstp = jnp.concatenate([dst, jnp.zeros((npad,), jnp.int32)])
    dst_scat = jnp.concatenate(
        [dst, jnp.full((npad,), N, jnp.int32)]).reshape(NW * CPT, CHUNK)
    Jp = jnp.concatenate([J_msg, jnp.zeros((npad, 1), _f32)], axis=0)

    Tin, Tout = _k1(x, b, W_l, attl, attr, W1T)
    G = _k2(Tin, Tout, srcp, dstp)
    H = _k3(G, Jp, W1T, b1, msg_W2, b2)
    acc = _k4(H, dst_scat)
    out = _k5(acc, x, msg_W3, b3, gru_Wih, bih, gru_Whh, bhh, gbias)
    return out
